# baseline (speedup 1.0000x reference)
import functools

import jax
import jax.numpy as jnp
from jax import lax
from jax.experimental import pallas as pl
from jax.experimental.pallas import tpu as pltpu

HEAD_DIM = 32
NUM_HEADS = 12
GP_NUM = 48
TOPK = 96
HPAD = 128
_SIGN = -2147483648

_DN_NT = (((1,), (1,)), ((), ()))
_DN_TN = (((0,), (0,)), ((), ()))
_DN_NN = (((1,), (0,)), ((), ()))


def _route_body(x_ref, wq_ref, gp_ref, qkv_ref, oh_ref, qmw_ref):
    xb = x_ref[0]
    n = xb.shape[0]
    qkv = lax.dot_general(xb, wq_ref[...], _DN_NT,
                          preferred_element_type=jnp.float32)
    qkv_ref[0] = qkv.astype(jnp.bfloat16)

    ones_col = jnp.ones((n, 1), jnp.float32)
    s_rows = []
    for h in range(NUM_HEADS):
        blk = qkv[:, h * HPAD:(h + 1) * HPAD]
        gpp = gp_ref[h]
        glT = lax.dot_general(gpp, blk, _DN_NT,
                              preferred_element_type=jnp.float32)
        gmaxT = jnp.max(glT, axis=0, keepdims=True)
        iota_s = lax.broadcasted_iota(jnp.int32, glT.shape, 0)
        gidxT = jnp.min(jnp.where(glT == gmaxT, iota_s, GP_NUM), axis=0,
                        keepdims=True)
        ohT = (iota_s == gidxT).astype(jnp.float32)
        oh_ref[0, h] = ohT.astype(jnp.bfloat16)

        cnt = lax.dot_general(ohT, ones_col, _DN_NN,
                              preferred_element_type=jnp.float32)
        qsum = lax.dot_general(ohT, blk, _DN_NN,
                               preferred_element_type=jnp.float32)
        qmean = qsum / jnp.maximum(cnt, 1e-8)
        colv = lax.broadcasted_iota(jnp.int32, qmean.shape, 1)
        qm_q = jnp.where(colv < HEAD_DIM, qmean, 0.0)
        a = pltpu.roll(qm_q, HEAD_DIM, 1)
        qmw = lax.dot_general(a, blk, _DN_NT,
                              preferred_element_type=jnp.float32)
        qmw_ref[0, h] = qmw


def _attn_body(qkv_ref, oh_ref, qmw_ref, wp_ref, out_ref):
    h = pl.program_id(1)
    scale = HEAD_DIM ** (-0.5)
    blk = qkv_ref[0]
    ohT = oh_ref[0, 0]
    qmw = qmw_ref[0, 0]

    u = lax.bitcast_convert_type(qmw, jnp.int32)
    s = jnp.where(u >= 0, u, u ^ jnp.int32(0x7FFFFFFF))
    c = jnp.zeros((GP_NUM, 1), jnp.int32)
    for i in range(32):
        cand = c | jnp.int32(1 << (31 - i)) if i else jnp.full(
            (GP_NUM, 1), _SIGN, jnp.int32)
        cand_s = cand ^ jnp.int32(_SIGN)
        cnt_ge = jnp.sum((s >= cand_s).astype(jnp.int32), axis=1,
                         keepdims=True)
        c = jnp.where(cnt_ge >= TOPK, cand, c)
    thr = c ^ jnp.int32(_SIGN)
    gmask = (s >= thr).astype(jnp.bfloat16)

    col = lax.broadcasted_iota(jnp.int32, blk.shape, 1)
    bq = jnp.where(col < HEAD_DIM, blk, jnp.bfloat16(0))
    bk = pltpu.roll(blk, HPAD - HEAD_DIM, 1)
    scores = lax.dot_general(bq, bk, _DN_NT,
                             preferred_element_type=jnp.float32) * scale
    e = jnp.exp(scores)
    fmask = lax.dot_general(ohT, gmask, _DN_TN,
                            preferred_element_type=jnp.float32)
    p = e * fmask
    denom = jnp.sum(p, axis=1, keepdims=True)
    attn = (p / jnp.maximum(denom, 1e-30)).astype(jnp.bfloat16)
    o = lax.dot_general(attn, blk, _DN_NN,
                        preferred_element_type=jnp.float32)
    wp = wp_ref[0]
    contrib = lax.dot_general(o.astype(jnp.bfloat16), wp, _DN_NN,
                              preferred_element_type=jnp.float32)

    @pl.when(h == 0)
    def _init():
        out_ref[0] = contrib

    @pl.when(h != 0)
    def _acc():
        out_ref[0] = out_ref[0] + contrib


@jax.jit
def kernel(x, W_qkv, W_proj, W_gp):
    b, hh, ww, dim = x.shape
    n = hh * ww
    x3 = x.reshape(b, n, dim)

    wqr = jnp.transpose(W_qkv.reshape(3, NUM_HEADS, HEAD_DIM, dim),
                        (1, 0, 2, 3)).reshape(NUM_HEADS, 3 * HEAD_DIM, dim)
    wq_pad = jnp.pad(wqr, ((0, 0), (0, HPAD - 3 * HEAD_DIM), (0, 0))
                     ).reshape(NUM_HEADS * HPAD, dim)
    gp_pad = jnp.pad(W_gp.reshape(NUM_HEADS, GP_NUM, HEAD_DIM),
                     ((0, 0), (0, 0), (0, HPAD - HEAD_DIM)))
    wp_h = jnp.transpose(W_proj.reshape(dim, NUM_HEADS, HEAD_DIM), (1, 2, 0))
    wp_pad = jnp.pad(wp_h, ((0, 0), (2 * HEAD_DIM, HPAD - 3 * HEAD_DIM),
                            (0, 0))).astype(jnp.bfloat16)

    qkv, oh, gm = pl.pallas_call(
        _route_body,
        grid=(b,),
        in_specs=[
            pl.BlockSpec((1, n, dim), lambda i: (i, 0, 0)),
            pl.BlockSpec((NUM_HEADS * HPAD, dim), lambda i: (0, 0)),
            pl.BlockSpec((NUM_HEADS, GP_NUM, HPAD), lambda i: (0, 0, 0)),
        ],
        out_specs=[
            pl.BlockSpec((1, n, NUM_HEADS * HPAD), lambda i: (i, 0, 0)),
            pl.BlockSpec((1, NUM_HEADS, GP_NUM, n), lambda i: (i, 0, 0, 0)),
            pl.BlockSpec((1, NUM_HEADS, GP_NUM, n), lambda i: (i, 0, 0, 0)),
        ],
        out_shape=[
            jax.ShapeDtypeStruct((b, n, NUM_HEADS * HPAD), jnp.bfloat16),
            jax.ShapeDtypeStruct((b, NUM_HEADS, GP_NUM, n), jnp.bfloat16),
            jax.ShapeDtypeStruct((b, NUM_HEADS, GP_NUM, n), jnp.float32),
        ],
        compiler_params=pltpu.CompilerParams(
            dimension_semantics=("arbitrary",),
        ),
    )(x3, wq_pad, gp_pad)

    out = pl.pallas_call(
        _attn_body,
        grid=(b, NUM_HEADS),
        in_specs=[
            pl.BlockSpec((1, n, HPAD), lambda i, j: (i, 0, j)),
            pl.BlockSpec((1, 1, GP_NUM, n), lambda i, j: (i, j, 0, 0)),
            pl.BlockSpec((1, 1, GP_NUM, n), lambda i, j: (i, j, 0, 0)),
            pl.BlockSpec((1, HPAD, dim), lambda i, j: (j, 0, 0)),
        ],
        out_specs=pl.BlockSpec((1, n, dim), lambda i, j: (i, 0, 0)),
        out_shape=jax.ShapeDtypeStruct((b, n, dim), jnp.float32),
        compiler_params=pltpu.CompilerParams(
            dimension_semantics=("arbitrary", "arbitrary"),
        ),
    )(qkv, oh, gm, wp_pad)
    return out.reshape(b, hh, ww, dim)


# speedup vs baseline: 1.4474x; 1.4474x over previous
import functools

import jax
import jax.numpy as jnp
from jax import lax
from jax.experimental import pallas as pl
from jax.experimental.pallas import tpu as pltpu

HEAD_DIM = 32
NUM_HEADS = 12
GP_NUM = 48
TOPK = 96
HPAD = 128
_SIGN = -2147483648

_DN_NT = (((1,), (1,)), ((), ()))
_DN_TN = (((0,), (0,)), ((), ()))
_DN_NN = (((1,), (0,)), ((), ()))


def _route_body(x_ref, wq_ref, gp_ref, qkv_ref, oh_ref, gmask_ref):
    xb = x_ref[0]
    n = xb.shape[0]
    qkv = lax.dot_general(xb, wq_ref[...], _DN_NT,
                          preferred_element_type=jnp.float32)
    qkv_ref[0] = qkv.astype(jnp.bfloat16)

    ones_col = jnp.ones((n, 1), jnp.float32)
    s_rows = []
    for h in range(NUM_HEADS):
        blk = qkv[:, h * HPAD:(h + 1) * HPAD]
        gpp = gp_ref[h]
        glT = lax.dot_general(gpp, blk, _DN_NT,
                              preferred_element_type=jnp.float32)
        gmaxT = jnp.max(glT, axis=0, keepdims=True)
        iota_s = lax.broadcasted_iota(jnp.int32, glT.shape, 0)
        gidxT = jnp.min(jnp.where(glT == gmaxT, iota_s, GP_NUM), axis=0,
                        keepdims=True)
        ohT = (iota_s == gidxT).astype(jnp.float32)
        oh_ref[0, h] = ohT.astype(jnp.bfloat16)

        cnt = lax.dot_general(ohT, ones_col, _DN_NN,
                              preferred_element_type=jnp.float32)
        qsum = lax.dot_general(ohT, blk, _DN_NN,
                               preferred_element_type=jnp.float32)
        qmean = qsum / jnp.maximum(cnt, 1e-8)
        colv = lax.broadcasted_iota(jnp.int32, qmean.shape, 1)
        qm_q = jnp.where(colv < HEAD_DIM, qmean, 0.0)
        a = pltpu.roll(qm_q, HEAD_DIM, 1)
        qmw = lax.dot_general(a, blk, _DN_NT,
                              preferred_element_type=jnp.float32)
        u = lax.bitcast_convert_type(qmw, jnp.int32)
        s_rows.append(jnp.where(u >= 0, u, u ^ jnp.int32(0x7FFFFFFF)))

    s = jnp.concatenate(s_rows, axis=0)
    ones_n = jnp.ones((n, 1), jnp.float32)

    def bit_step(i, c):
        b = 31 - i
        cand = c | lax.shift_left(jnp.int32(1), b)
        cand_s = cand ^ jnp.int32(_SIGN)
        ge = (s >= cand_s).astype(jnp.float32)
        cnt_ge = lax.dot_general(ge, ones_n, _DN_NN,
                                 preferred_element_type=jnp.float32)
        return jnp.where(cnt_ge >= float(TOPK), cand, c)

    c = lax.fori_loop(0, 32, bit_step,
                      jnp.zeros((NUM_HEADS * GP_NUM, 1), jnp.int32))
    thr = c ^ jnp.int32(_SIGN)
    gmask = (s >= thr).astype(jnp.bfloat16)
    for h in range(NUM_HEADS):
        gmask_ref[0, h] = gmask[h * GP_NUM:(h + 1) * GP_NUM, :]


def _attn_body(qkv_ref, oh_ref, gm_ref, wp_ref, out_ref):
    h = pl.program_id(1)
    scale = HEAD_DIM ** (-0.5)
    blk = qkv_ref[0]
    ohT = oh_ref[0, 0]
    gmask = gm_ref[0, 0]

    col = lax.broadcasted_iota(jnp.int32, blk.shape, 1)
    bq = jnp.where(col < HEAD_DIM, blk, jnp.bfloat16(0))
    bk = pltpu.roll(blk, HPAD - HEAD_DIM, 1)
    scores = lax.dot_general(bq, bk, _DN_NT,
                             preferred_element_type=jnp.float32) * scale
    e = jnp.exp(scores)
    fmask = lax.dot_general(ohT, gmask, _DN_TN,
                            preferred_element_type=jnp.float32)
    p = e * fmask
    denom = jnp.sum(p, axis=1, keepdims=True)
    attn = (p / jnp.maximum(denom, 1e-30)).astype(jnp.bfloat16)
    o = lax.dot_general(attn, blk, _DN_NN,
                        preferred_element_type=jnp.float32)
    wp = wp_ref[0]
    contrib = lax.dot_general(o.astype(jnp.bfloat16), wp, _DN_NN,
                              preferred_element_type=jnp.float32)

    @pl.when(h == 0)
    def _init():
        out_ref[0] = contrib

    @pl.when(h != 0)
    def _acc():
        out_ref[0] = out_ref[0] + contrib


@jax.jit
def kernel(x, W_qkv, W_proj, W_gp):
    b, hh, ww, dim = x.shape
    n = hh * ww
    x3 = x.reshape(b, n, dim)

    wqr = jnp.transpose(W_qkv.reshape(3, NUM_HEADS, HEAD_DIM, dim),
                        (1, 0, 2, 3)).reshape(NUM_HEADS, 3 * HEAD_DIM, dim)
    wq_pad = jnp.pad(wqr, ((0, 0), (0, HPAD - 3 * HEAD_DIM), (0, 0))
                     ).reshape(NUM_HEADS * HPAD, dim)
    gp_pad = jnp.pad(W_gp.reshape(NUM_HEADS, GP_NUM, HEAD_DIM),
                     ((0, 0), (0, 0), (0, HPAD - HEAD_DIM)))
    wp_h = jnp.transpose(W_proj.reshape(dim, NUM_HEADS, HEAD_DIM), (1, 2, 0))
    wp_pad = jnp.pad(wp_h, ((0, 0), (2 * HEAD_DIM, HPAD - 3 * HEAD_DIM),
                            (0, 0))).astype(jnp.bfloat16)

    qkv, oh, gm = pl.pallas_call(
        _route_body,
        grid=(b,),
        in_specs=[
            pl.BlockSpec((1, n, dim), lambda i: (i, 0, 0)),
            pl.BlockSpec((NUM_HEADS * HPAD, dim), lambda i: (0, 0)),
            pl.BlockSpec((NUM_HEADS, GP_NUM, HPAD), lambda i: (0, 0, 0)),
        ],
        out_specs=[
            pl.BlockSpec((1, n, NUM_HEADS * HPAD), lambda i: (i, 0, 0)),
            pl.BlockSpec((1, NUM_HEADS, GP_NUM, n), lambda i: (i, 0, 0, 0)),
            pl.BlockSpec((1, NUM_HEADS, GP_NUM, n), lambda i: (i, 0, 0, 0)),
        ],
        out_shape=[
            jax.ShapeDtypeStruct((b, n, NUM_HEADS * HPAD), jnp.bfloat16),
            jax.ShapeDtypeStruct((b, NUM_HEADS, GP_NUM, n), jnp.bfloat16),
            jax.ShapeDtypeStruct((b, NUM_HEADS, GP_NUM, n), jnp.bfloat16),
        ],
        compiler_params=pltpu.CompilerParams(
            dimension_semantics=("arbitrary",),
        ),
    )(x3, wq_pad, gp_pad)

    out = pl.pallas_call(
        _attn_body,
        grid=(b, NUM_HEADS),
        in_specs=[
            pl.BlockSpec((1, n, HPAD), lambda i, j: (i, 0, j)),
            pl.BlockSpec((1, 1, GP_NUM, n), lambda i, j: (i, j, 0, 0)),
            pl.BlockSpec((1, 1, GP_NUM, n), lambda i, j: (i, j, 0, 0)),
            pl.BlockSpec((1, HPAD, dim), lambda i, j: (j, 0, 0)),
        ],
        out_specs=pl.BlockSpec((1, n, dim), lambda i, j: (i, 0, 0)),
        out_shape=jax.ShapeDtypeStruct((b, n, dim), jnp.float32),
        compiler_params=pltpu.CompilerParams(
            dimension_semantics=("arbitrary", "arbitrary"),
        ),
    )(qkv, oh, gm, wp_pad)
    return out.reshape(b, hh, ww, dim)


# speedup vs baseline: 1.6833x; 1.1630x over previous
import functools

import jax
import jax.numpy as jnp
from jax import lax
from jax.experimental import pallas as pl
from jax.experimental.pallas import tpu as pltpu

HEAD_DIM = 32
NUM_HEADS = 12
GP_NUM = 48
TOPK = 96
HPAD = 128
_SIGN = -2147483648

_DN_NT = (((1,), (1,)), ((), ()))
_DN_TN = (((0,), (0,)), ((), ()))
_DN_NN = (((1,), (0,)), ((), ()))


def _route_body(nb, x_ref, wq_ref, gp_ref, qkv_ref, oh_ref, gmask_ref,
                s_scr):
    i = pl.program_id(0)

    @pl.when(i < nb)
    def _route():
        xb = x_ref[0]
        n = xb.shape[0]
        qkv = lax.dot_general(xb, wq_ref[...], _DN_NT,
                              preferred_element_type=jnp.float32)
        qkv_ref[0] = qkv.astype(jnp.bfloat16)

        ones_col = jnp.ones((n, 1), jnp.float32)
        s_rows = []
        for h in range(NUM_HEADS):
            blk = qkv[:, h * HPAD:(h + 1) * HPAD]
            gpp = gp_ref[h]
            glT = lax.dot_general(gpp, blk, _DN_NT,
                                  preferred_element_type=jnp.float32)
            gmaxT = jnp.max(glT, axis=0, keepdims=True)
            iota_s = lax.broadcasted_iota(jnp.int32, glT.shape, 0)
            gidxT = jnp.min(jnp.where(glT == gmaxT, iota_s, GP_NUM), axis=0,
                            keepdims=True)
            ohT = (iota_s == gidxT).astype(jnp.float32)
            oh_ref[0, h] = ohT.astype(jnp.bfloat16)

            cnt = lax.dot_general(ohT, ones_col, _DN_NN,
                                  preferred_element_type=jnp.float32)
            qsum = lax.dot_general(ohT, blk, _DN_NN,
                                   preferred_element_type=jnp.float32)
            qmean = qsum / jnp.maximum(cnt, 1e-8)
            colv = lax.broadcasted_iota(jnp.int32, qmean.shape, 1)
            qm_q = jnp.where(colv < HEAD_DIM, qmean, 0.0)
            a = pltpu.roll(qm_q, HEAD_DIM, 1)
            qmw = lax.dot_general(a, blk, _DN_NT,
                                  preferred_element_type=jnp.float32)
            u = lax.bitcast_convert_type(qmw, jnp.int32)
            s_rows.append(jnp.where(u >= 0, u, u ^ jnp.int32(0x7FFFFFFF)))

        s_scr[lax.rem(i, 2)] = jnp.concatenate(s_rows, axis=0)

    @pl.when(i > 0)
    def _search():
        s = s_scr[lax.rem(i + 1, 2)]
        c = jnp.zeros((NUM_HEADS * GP_NUM, 1), jnp.int32)
        for it in range(32):
            if it == 0:
                cand = jnp.full((NUM_HEADS * GP_NUM, 1), _SIGN, jnp.int32)
            else:
                cand = c | jnp.int32(1 << (31 - it))
            cand_s = cand ^ jnp.int32(_SIGN)
            cnt_ge = jnp.sum((s >= cand_s).astype(jnp.int32), axis=1,
                             keepdims=True)
            c = jnp.where(cnt_ge >= TOPK, cand, c)
        thr = c ^ jnp.int32(_SIGN)
        gmask = (s >= thr).astype(jnp.bfloat16)
        for h in range(NUM_HEADS):
            gmask_ref[0, h] = gmask[h * GP_NUM:(h + 1) * GP_NUM, :]


def _attn_body(qkv_ref, oh_ref, gm_ref, wp_ref, out_ref):
    h = pl.program_id(1)
    scale = HEAD_DIM ** (-0.5)
    blk = qkv_ref[0]
    ohT = oh_ref[0, 0]
    gmask = gm_ref[0, 0]

    col = lax.broadcasted_iota(jnp.int32, blk.shape, 1)
    bq = jnp.where(col < HEAD_DIM, blk, jnp.bfloat16(0))
    bk = pltpu.roll(blk, HPAD - HEAD_DIM, 1)
    scores = lax.dot_general(bq, bk, _DN_NT,
                             preferred_element_type=jnp.float32) * scale
    e = jnp.exp(scores)
    fmask = lax.dot_general(ohT, gmask, _DN_TN,
                            preferred_element_type=jnp.float32)
    p = e * fmask
    denom = jnp.sum(p, axis=1, keepdims=True)
    attn = (p / jnp.maximum(denom, 1e-30)).astype(jnp.bfloat16)
    o = lax.dot_general(attn, blk, _DN_NN,
                        preferred_element_type=jnp.float32)
    wp = wp_ref[0]
    contrib = lax.dot_general(o.astype(jnp.bfloat16), wp, _DN_NN,
                              preferred_element_type=jnp.float32)

    @pl.when(h == 0)
    def _init():
        out_ref[0] = contrib

    @pl.when(h != 0)
    def _acc():
        out_ref[0] = out_ref[0] + contrib


@jax.jit
def kernel(x, W_qkv, W_proj, W_gp):
    b, hh, ww, dim = x.shape
    n = hh * ww
    x3 = x.reshape(b, n, dim)

    wqr = jnp.transpose(W_qkv.reshape(3, NUM_HEADS, HEAD_DIM, dim),
                        (1, 0, 2, 3)).reshape(NUM_HEADS, 3 * HEAD_DIM, dim)
    wq_pad = jnp.pad(wqr, ((0, 0), (0, HPAD - 3 * HEAD_DIM), (0, 0))
                     ).reshape(NUM_HEADS * HPAD, dim)
    gp_pad = jnp.pad(W_gp.reshape(NUM_HEADS, GP_NUM, HEAD_DIM),
                     ((0, 0), (0, 0), (0, HPAD - HEAD_DIM)))
    wp_h = jnp.transpose(W_proj.reshape(dim, NUM_HEADS, HEAD_DIM), (1, 2, 0))
    wp_pad = jnp.pad(wp_h, ((0, 0), (2 * HEAD_DIM, HPAD - 3 * HEAD_DIM),
                            (0, 0))).astype(jnp.bfloat16)

    qkv, oh, gm = pl.pallas_call(
        functools.partial(_route_body, b),
        grid=(b + 1,),
        in_specs=[
            pl.BlockSpec((1, n, dim), lambda i: (jnp.minimum(i, b - 1), 0, 0)),
            pl.BlockSpec((NUM_HEADS * HPAD, dim), lambda i: (0, 0)),
            pl.BlockSpec((NUM_HEADS, GP_NUM, HPAD), lambda i: (0, 0, 0)),
        ],
        out_specs=[
            pl.BlockSpec((1, n, NUM_HEADS * HPAD),
                         lambda i: (jnp.minimum(i, b - 1), 0, 0)),
            pl.BlockSpec((1, NUM_HEADS, GP_NUM, n),
                         lambda i: (jnp.minimum(i, b - 1), 0, 0, 0)),
            pl.BlockSpec((1, NUM_HEADS, GP_NUM, n),
                         lambda i: (jnp.maximum(i - 1, 0), 0, 0, 0)),
        ],
        out_shape=[
            jax.ShapeDtypeStruct((b, n, NUM_HEADS * HPAD), jnp.bfloat16),
            jax.ShapeDtypeStruct((b, NUM_HEADS, GP_NUM, n), jnp.bfloat16),
            jax.ShapeDtypeStruct((b, NUM_HEADS, GP_NUM, n), jnp.bfloat16),
        ],
        scratch_shapes=[
            pltpu.VMEM((2, NUM_HEADS * GP_NUM, n), jnp.int32),
        ],
        compiler_params=pltpu.CompilerParams(
            dimension_semantics=("arbitrary",),
        ),
    )(x3, wq_pad, gp_pad)

    out = pl.pallas_call(
        _attn_body,
        grid=(b, NUM_HEADS),
        in_specs=[
            pl.BlockSpec((1, n, HPAD), lambda i, j: (i, 0, j)),
            pl.BlockSpec((1, 1, GP_NUM, n), lambda i, j: (i, j, 0, 0)),
            pl.BlockSpec((1, 1, GP_NUM, n), lambda i, j: (i, j, 0, 0)),
            pl.BlockSpec((1, HPAD, dim), lambda i, j: (j, 0, 0)),
        ],
        out_specs=pl.BlockSpec((1, n, dim), lambda i, j: (i, 0, 0)),
        out_shape=jax.ShapeDtypeStruct((b, n, dim), jnp.float32),
        compiler_params=pltpu.CompilerParams(
            dimension_semantics=("arbitrary", "arbitrary"),
        ),
    )(qkv, oh, gm, wp_pad)
    return out.reshape(b, hh, ww, dim)


# speedup vs baseline: 1.8099x; 1.0752x over previous
import functools

import jax
import jax.numpy as jnp
from jax import lax
from jax.experimental import pallas as pl
from jax.experimental.pallas import tpu as pltpu

HEAD_DIM = 32
NUM_HEADS = 12
GP_NUM = 48
TOPK = 96
HPAD = 128
_SIGN = -2147483648

_DN_NT = (((1,), (1,)), ((), ()))
_DN_TN = (((0,), (0,)), ((), ()))
_DN_NN = (((1,), (0,)), ((), ()))


def _route_body(nb, x_ref, wq_ref, gp_ref, qkv_ref, oh_ref, gmask_ref,
                s_scr):
    i = pl.program_id(0)
    del nb

    if True:
        xb = x_ref[0]
        n = xb.shape[0]
        qkv = lax.dot_general(xb, wq_ref[...], _DN_NT,
                              preferred_element_type=jnp.float32)
        qkv_ref[0] = qkv.astype(jnp.bfloat16)

        ones_col = jnp.ones((n, 1), jnp.float32)
        s_rows = []
        for h in range(NUM_HEADS):
            blk = qkv[:, h * HPAD:(h + 1) * HPAD]
            gpp = gp_ref[h]
            glT = lax.dot_general(gpp, blk, _DN_NT,
                                  preferred_element_type=jnp.float32)
            gmaxT = jnp.max(glT, axis=0, keepdims=True)
            iota_s = lax.broadcasted_iota(jnp.int32, glT.shape, 0)
            gidxT = jnp.min(jnp.where(glT == gmaxT, iota_s, GP_NUM), axis=0,
                            keepdims=True)
            ohT = (iota_s == gidxT).astype(jnp.float32)
            oh_ref[0, h] = ohT.astype(jnp.bfloat16)

            cnt = lax.dot_general(ohT, ones_col, _DN_NN,
                                  preferred_element_type=jnp.float32)
            qsum = lax.dot_general(ohT, blk, _DN_NN,
                                   preferred_element_type=jnp.float32)
            qmean = qsum / jnp.maximum(cnt, 1e-8)
            colv = lax.broadcasted_iota(jnp.int32, qmean.shape, 1)
            qm_q = jnp.where(colv < HEAD_DIM, qmean, 0.0)
            a = pltpu.roll(qm_q, HEAD_DIM, 1)
            qmw = lax.dot_general(a, blk, _DN_NT,
                                  preferred_element_type=jnp.float32)
            u = lax.bitcast_convert_type(qmw, jnp.int32)
            s_rows.append(jnp.where(u >= 0, u, u ^ jnp.int32(0x7FFFFFFF)))

        s_scr[lax.rem(i, 2)] = jnp.concatenate(s_rows, axis=0)

    if True:
        s = s_scr[lax.rem(i + 1, 2)]
        c = jnp.zeros((NUM_HEADS * GP_NUM, 1), jnp.int32)
        for it in range(32):
            if it == 0:
                cand = jnp.full((NUM_HEADS * GP_NUM, 1), _SIGN, jnp.int32)
            else:
                cand = c | jnp.int32(1 << (31 - it))
            cand_s = cand ^ jnp.int32(_SIGN)
            cnt_ge = jnp.sum((s >= cand_s).astype(jnp.int32), axis=1,
                             keepdims=True)
            c = jnp.where(cnt_ge >= TOPK, cand, c)
        thr = c ^ jnp.int32(_SIGN)
        gmask = (s >= thr).astype(jnp.bfloat16)
        for h in range(NUM_HEADS):
            gmask_ref[0, h] = gmask[h * GP_NUM:(h + 1) * GP_NUM, :]


def _attn_body(qkv_ref, oh_ref, gm_ref, wp_ref, out_ref):
    h = pl.program_id(1)
    scale = HEAD_DIM ** (-0.5)
    blk = qkv_ref[0]
    ohT = oh_ref[0, 0]
    gmask = gm_ref[0, 0]

    col = lax.broadcasted_iota(jnp.int32, blk.shape, 1)
    bq = jnp.where(col < HEAD_DIM, blk, jnp.bfloat16(0))
    bk = pltpu.roll(blk, HPAD - HEAD_DIM, 1)
    scores = lax.dot_general(bq, bk, _DN_NT,
                             preferred_element_type=jnp.float32) * scale
    e = jnp.exp(scores)
    fmask = lax.dot_general(ohT, gmask, _DN_TN,
                            preferred_element_type=jnp.float32)
    p = e * fmask
    denom = jnp.sum(p, axis=1, keepdims=True)
    attn = (p / jnp.maximum(denom, 1e-30)).astype(jnp.bfloat16)
    o = lax.dot_general(attn, blk, _DN_NN,
                        preferred_element_type=jnp.float32)
    wp = wp_ref[0]
    contrib = lax.dot_general(o.astype(jnp.bfloat16), wp, _DN_NN,
                              preferred_element_type=jnp.float32)

    @pl.when(h == 0)
    def _init():
        out_ref[0] = contrib

    @pl.when(h != 0)
    def _acc():
        out_ref[0] = out_ref[0] + contrib


@jax.jit
def kernel(x, W_qkv, W_proj, W_gp):
    b, hh, ww, dim = x.shape
    n = hh * ww
    x3 = x.reshape(b, n, dim)

    wqr = jnp.transpose(W_qkv.reshape(3, NUM_HEADS, HEAD_DIM, dim),
                        (1, 0, 2, 3)).reshape(NUM_HEADS, 3 * HEAD_DIM, dim)
    wq_pad = jnp.pad(wqr, ((0, 0), (0, HPAD - 3 * HEAD_DIM), (0, 0))
                     ).reshape(NUM_HEADS * HPAD, dim)
    gp_pad = jnp.pad(W_gp.reshape(NUM_HEADS, GP_NUM, HEAD_DIM),
                     ((0, 0), (0, 0), (0, HPAD - HEAD_DIM)))
    wp_h = jnp.transpose(W_proj.reshape(dim, NUM_HEADS, HEAD_DIM), (1, 2, 0))
    wp_pad = jnp.pad(wp_h, ((0, 0), (2 * HEAD_DIM, HPAD - 3 * HEAD_DIM),
                            (0, 0))).astype(jnp.bfloat16)

    qkv, oh, gm = pl.pallas_call(
        functools.partial(_route_body, b),
        grid=(b + 1,),
        in_specs=[
            pl.BlockSpec((1, n, dim), lambda i: (jnp.minimum(i, b - 1), 0, 0)),
            pl.BlockSpec((NUM_HEADS * HPAD, dim), lambda i: (0, 0)),
            pl.BlockSpec((NUM_HEADS, GP_NUM, HPAD), lambda i: (0, 0, 0)),
        ],
        out_specs=[
            pl.BlockSpec((1, n, NUM_HEADS * HPAD),
                         lambda i: (jnp.minimum(i, b - 1), 0, 0)),
            pl.BlockSpec((1, NUM_HEADS, GP_NUM, n),
                         lambda i: (jnp.minimum(i, b - 1), 0, 0, 0)),
            pl.BlockSpec((1, NUM_HEADS, GP_NUM, n),
                         lambda i: (jnp.maximum(i - 1, 0), 0, 0, 0)),
        ],
        out_shape=[
            jax.ShapeDtypeStruct((b, n, NUM_HEADS * HPAD), jnp.bfloat16),
            jax.ShapeDtypeStruct((b, NUM_HEADS, GP_NUM, n), jnp.bfloat16),
            jax.ShapeDtypeStruct((b, NUM_HEADS, GP_NUM, n), jnp.bfloat16),
        ],
        scratch_shapes=[
            pltpu.VMEM((2, NUM_HEADS * GP_NUM, n), jnp.int32),
        ],
        compiler_params=pltpu.CompilerParams(
            dimension_semantics=("arbitrary",),
        ),
    )(x3, wq_pad, gp_pad)

    out = pl.pallas_call(
        _attn_body,
        grid=(b, NUM_HEADS),
        in_specs=[
            pl.BlockSpec((1, n, HPAD), lambda i, j: (i, 0, j)),
            pl.BlockSpec((1, 1, GP_NUM, n), lambda i, j: (i, j, 0, 0)),
            pl.BlockSpec((1, 1, GP_NUM, n), lambda i, j: (i, j, 0, 0)),
            pl.BlockSpec((1, HPAD, dim), lambda i, j: (j, 0, 0)),
        ],
        out_specs=pl.BlockSpec((1, n, dim), lambda i, j: (i, 0, 0)),
        out_shape=jax.ShapeDtypeStruct((b, n, dim), jnp.float32),
        compiler_params=pltpu.CompilerParams(
            dimension_semantics=("arbitrary", "arbitrary"),
        ),
    )(qkv, oh, gm, wp_pad)
    return out.reshape(b, hh, ww, dim)


# speedup vs baseline: 1.8907x; 1.0446x over previous
import functools

import jax
import jax.numpy as jnp
from jax import lax
from jax.experimental import pallas as pl
from jax.experimental.pallas import tpu as pltpu

HEAD_DIM = 32
NUM_HEADS = 12
GP_NUM = 48
TOPK = 96
HPAD = 128
_SIGN = -2147483648

_DN_NT = (((1,), (1,)), ((), ()))
_DN_TN = (((0,), (0,)), ((), ()))
_DN_NN = (((1,), (0,)), ((), ()))


def _route_body(nb, x_ref, wq_ref, gp_ref, qkv_ref, oh_ref, gmask_ref,
                s_scr):
    i = pl.program_id(0)
    del nb

    if True:
        xb = x_ref[0]
        n = xb.shape[0]
        qkv = lax.dot_general(xb, wq_ref[...], _DN_NT,
                              preferred_element_type=jnp.float32)
        qkv_ref[0] = qkv.astype(jnp.bfloat16)

        ones_col = jnp.ones((n, 1), jnp.float32)
        s_rows = []
        for h in range(NUM_HEADS):
            blk = qkv[:, h * HPAD:(h + 1) * HPAD]
            gpp = gp_ref[h]
            glT = lax.dot_general(gpp, blk, _DN_NT,
                                  preferred_element_type=jnp.float32)
            gmaxT = jnp.max(glT, axis=0, keepdims=True)
            iota_s = lax.broadcasted_iota(jnp.int32, glT.shape, 0)
            gidxT = jnp.min(jnp.where(glT == gmaxT, iota_s, GP_NUM), axis=0,
                            keepdims=True)
            ohT = (iota_s == gidxT).astype(jnp.float32)
            oh_ref[0, h] = ohT.astype(jnp.bfloat16)

            cnt = lax.dot_general(ohT, ones_col, _DN_NN,
                                  preferred_element_type=jnp.float32)
            qsum = lax.dot_general(ohT, blk, _DN_NN,
                                   preferred_element_type=jnp.float32)
            qmean = qsum / jnp.maximum(cnt, 1e-8)
            colv = lax.broadcasted_iota(jnp.int32, qmean.shape, 1)
            qm_q = jnp.where(colv < HEAD_DIM, qmean, 0.0)
            a = pltpu.roll(qm_q, HEAD_DIM, 1)
            qmw = lax.dot_general(a, blk, _DN_NT,
                                  preferred_element_type=jnp.float32)
            u = lax.bitcast_convert_type(qmw, jnp.int32)
            s_rows.append(jnp.where(u >= 0, u, u ^ jnp.int32(0x7FFFFFFF)))

        s_scr[lax.rem(i, 2)] = jnp.concatenate(s_rows, axis=0)

    if True:
        s = s_scr[lax.rem(i + 1, 2)]
        c = jnp.zeros((NUM_HEADS * GP_NUM, 1), jnp.int32)
        for it in range(32):
            if it == 0:
                cand = jnp.full((NUM_HEADS * GP_NUM, 1), _SIGN, jnp.int32)
            else:
                cand = c | jnp.int32(1 << (31 - it))
            cand_s = cand ^ jnp.int32(_SIGN)
            cnt_ge = jnp.sum((s >= cand_s).astype(jnp.int32), axis=1,
                             keepdims=True)
            c = jnp.where(cnt_ge >= TOPK, cand, c)
        thr = c ^ jnp.int32(_SIGN)
        gmask = (s >= thr).astype(jnp.bfloat16)
        for h in range(NUM_HEADS):
            gmask_ref[0, h] = gmask[h * GP_NUM:(h + 1) * GP_NUM, :]


def _attn_body(qkv_ref, oh_ref, gm_ref, wp_ref, out_ref):
    h = pl.program_id(1)
    scale = HEAD_DIM ** (-0.5)
    blk = qkv_ref[0]
    ohT = oh_ref[0, 0]
    gmask = gm_ref[0, 0]

    col = lax.broadcasted_iota(jnp.int32, blk.shape, 1)
    bq = jnp.where(col < HEAD_DIM, blk, jnp.bfloat16(0))
    bk = pltpu.roll(blk, HPAD - HEAD_DIM, 1)
    wp = wp_ref[0]

    scores = lax.dot_general(bq, bk, _DN_NT,
                             preferred_element_type=jnp.float32) * scale
    e = jnp.exp(scores)
    fmask = lax.dot_general(ohT, gmask, _DN_TN,
                            preferred_element_type=jnp.float32)
    pf = e * fmask
    p = pf.astype(jnp.bfloat16)
    denom = jnp.sum(pf, axis=1, keepdims=True)
    o = lax.dot_general(p, blk, _DN_NN,
                        preferred_element_type=jnp.float32)
    o = o / jnp.maximum(denom, 1e-30)
    contrib = lax.dot_general(o.astype(jnp.bfloat16), wp, _DN_NN,
                              preferred_element_type=jnp.float32)

    @pl.when(h == 0)
    def _init():
        out_ref[0] = contrib

    @pl.when(h != 0)
    def _acc():
        out_ref[0] = out_ref[0] + contrib


@jax.jit
def kernel(x, W_qkv, W_proj, W_gp):
    b, hh, ww, dim = x.shape
    n = hh * ww
    x3 = x.reshape(b, n, dim)

    wqr = jnp.transpose(W_qkv.reshape(3, NUM_HEADS, HEAD_DIM, dim),
                        (1, 0, 2, 3)).reshape(NUM_HEADS, 3 * HEAD_DIM, dim)
    wq_pad = jnp.pad(wqr, ((0, 0), (0, HPAD - 3 * HEAD_DIM), (0, 0))
                     ).reshape(NUM_HEADS * HPAD, dim)
    gp_pad = jnp.pad(W_gp.reshape(NUM_HEADS, GP_NUM, HEAD_DIM),
                     ((0, 0), (0, 0), (0, HPAD - HEAD_DIM)))
    wp_h = jnp.transpose(W_proj.reshape(dim, NUM_HEADS, HEAD_DIM), (1, 2, 0))
    wp_pad = jnp.pad(wp_h, ((0, 0), (2 * HEAD_DIM, HPAD - 3 * HEAD_DIM),
                            (0, 0))).astype(jnp.bfloat16)

    qkv, oh, gm = pl.pallas_call(
        functools.partial(_route_body, b),
        grid=(b + 1,),
        in_specs=[
            pl.BlockSpec((1, n, dim), lambda i: (jnp.minimum(i, b - 1), 0, 0)),
            pl.BlockSpec((NUM_HEADS * HPAD, dim), lambda i: (0, 0)),
            pl.BlockSpec((NUM_HEADS, GP_NUM, HPAD), lambda i: (0, 0, 0)),
        ],
        out_specs=[
            pl.BlockSpec((1, n, NUM_HEADS * HPAD),
                         lambda i: (jnp.minimum(i, b - 1), 0, 0)),
            pl.BlockSpec((1, NUM_HEADS, GP_NUM, n),
                         lambda i: (jnp.minimum(i, b - 1), 0, 0, 0)),
            pl.BlockSpec((1, NUM_HEADS, GP_NUM, n),
                         lambda i: (jnp.maximum(i - 1, 0), 0, 0, 0)),
        ],
        out_shape=[
            jax.ShapeDtypeStruct((b, n, NUM_HEADS * HPAD), jnp.bfloat16),
            jax.ShapeDtypeStruct((b, NUM_HEADS, GP_NUM, n), jnp.bfloat16),
            jax.ShapeDtypeStruct((b, NUM_HEADS, GP_NUM, n), jnp.bfloat16),
        ],
        scratch_shapes=[
            pltpu.VMEM((2, NUM_HEADS * GP_NUM, n), jnp.int32),
        ],
        compiler_params=pltpu.CompilerParams(
            dimension_semantics=("arbitrary",),
        ),
    )(x3, wq_pad, gp_pad)

    out = pl.pallas_call(
        _attn_body,
        grid=(b, NUM_HEADS),
        in_specs=[
            pl.BlockSpec((1, n, HPAD), lambda i, j: (i, 0, j)),
            pl.BlockSpec((1, 1, GP_NUM, n), lambda i, j: (i, j, 0, 0)),
            pl.BlockSpec((1, 1, GP_NUM, n), lambda i, j: (i, j, 0, 0)),
            pl.BlockSpec((1, HPAD, dim), lambda i, j: (j, 0, 0)),
        ],
        out_specs=pl.BlockSpec((1, n, dim), lambda i, j: (i, 0, 0)),
        out_shape=jax.ShapeDtypeStruct((b, n, dim), jnp.float32),
        compiler_params=pltpu.CompilerParams(
            dimension_semantics=("arbitrary", "arbitrary"),
        ),
    )(qkv, oh, gm, wp_pad)
    return out.reshape(b, hh, ww, dim)


# speedup vs baseline: 1.9191x; 1.0150x over previous
import functools

import jax
import jax.numpy as jnp
from jax import lax
from jax.experimental import pallas as pl
from jax.experimental.pallas import tpu as pltpu

HEAD_DIM = 32
NUM_HEADS = 12
GP_NUM = 48
TOPK = 96
HPAD = 128
_SIGN = -2147483648

_DN_NT = (((1,), (1,)), ((), ()))
_DN_TN = (((0,), (0,)), ((), ()))
_DN_NN = (((1,), (0,)), ((), ()))


def _route_body(nb, x_ref, wq_ref, gp_ref, qkv_ref, oh_ref, gmask_ref,
                s_scr):
    i = pl.program_id(0)
    del nb

    if True:
        xb = x_ref[0]
        n = xb.shape[0]
        qkv = lax.dot_general(xb, wq_ref[...], _DN_NT,
                              preferred_element_type=jnp.float32)
        qkv_ref[0] = qkv.astype(jnp.bfloat16)

        ones_col = jnp.ones((n, 1), jnp.float32)
        s_rows = []
        for h in range(NUM_HEADS):
            blk = qkv[:, h * HPAD:(h + 1) * HPAD]
            gpp = gp_ref[h]
            glT = lax.dot_general(gpp, blk, _DN_NT,
                                  preferred_element_type=jnp.float32)
            gmaxT = jnp.max(glT, axis=0, keepdims=True)
            iota_s = lax.broadcasted_iota(jnp.int32, glT.shape, 0)
            gidxT = jnp.min(jnp.where(glT == gmaxT, iota_s, GP_NUM), axis=0,
                            keepdims=True)
            ohT = (iota_s == gidxT).astype(jnp.float32)
            oh_ref[0, h] = ohT.astype(jnp.bfloat16)

            cnt = lax.dot_general(ohT, ones_col, _DN_NN,
                                  preferred_element_type=jnp.float32)
            qsum = lax.dot_general(ohT, blk, _DN_NN,
                                   preferred_element_type=jnp.float32)
            qmean = qsum / jnp.maximum(cnt, 1e-8)
            colv = lax.broadcasted_iota(jnp.int32, qmean.shape, 1)
            qm_q = jnp.where(colv < HEAD_DIM, qmean, 0.0)
            a = pltpu.roll(qm_q, HEAD_DIM, 1)
            qmw = lax.dot_general(a, blk, _DN_NT,
                                  preferred_element_type=jnp.float32)
            u = lax.bitcast_convert_type(qmw, jnp.int32)
            s_rows.append(jnp.where(u >= 0, u, u ^ jnp.int32(0x7FFFFFFF)))

        s_scr[lax.rem(i, 2)] = jnp.concatenate(s_rows, axis=0)

    if True:
        s = s_scr[lax.rem(i + 1, 2)]
        c = jnp.zeros((NUM_HEADS * GP_NUM, 1), jnp.int32)
        for it in range(32):
            if it == 0:
                cand = jnp.full((NUM_HEADS * GP_NUM, 1), _SIGN, jnp.int32)
            else:
                cand = c | jnp.int32(1 << (31 - it))
            cand_s = cand ^ jnp.int32(_SIGN)
            cnt_ge = jnp.sum((s >= cand_s).astype(jnp.int32), axis=1,
                             keepdims=True)
            c = jnp.where(cnt_ge >= TOPK, cand, c)
        thr = c ^ jnp.int32(_SIGN)
        gmask = (s >= thr).astype(jnp.bfloat16)
        for h in range(NUM_HEADS):
            gmask_ref[0, h] = gmask[h * GP_NUM:(h + 1) * GP_NUM, :]


def _attn_body(qkv_ref, oh_ref, gm_ref, wp_ref, out_ref):
    h = pl.program_id(1)
    scale = HEAD_DIM ** (-0.5)
    blk = qkv_ref[0]
    ohT = oh_ref[0, 0]
    gmask = gm_ref[0, 0]

    col = lax.broadcasted_iota(jnp.int32, blk.shape, 1)
    bq = jnp.where(col < HEAD_DIM, blk * jnp.bfloat16(scale), jnp.bfloat16(0))
    bk = pltpu.roll(blk, HPAD - HEAD_DIM, 1)
    wp = wp_ref[0]

    scores = lax.dot_general(bq, bk, _DN_NT,
                             preferred_element_type=jnp.float32)
    e = jnp.exp(scores)
    fmask = lax.dot_general(ohT, gmask, _DN_TN,
                            preferred_element_type=jnp.float32)
    pf = e * fmask
    p = pf.astype(jnp.bfloat16)
    denom = jnp.sum(pf, axis=1, keepdims=True)
    o = lax.dot_general(p, blk, _DN_NN,
                        preferred_element_type=jnp.float32)
    o = o / jnp.maximum(denom, 1e-30)
    contrib = lax.dot_general(o.astype(jnp.bfloat16), wp, _DN_NN,
                              preferred_element_type=jnp.float32)

    @pl.when(h == 0)
    def _init():
        out_ref[0] = contrib

    @pl.when(h != 0)
    def _acc():
        out_ref[0] = out_ref[0] + contrib


@jax.jit
def kernel(x, W_qkv, W_proj, W_gp):
    b, hh, ww, dim = x.shape
    n = hh * ww
    x3 = x.reshape(b, n, dim)

    wqr = jnp.transpose(W_qkv.reshape(3, NUM_HEADS, HEAD_DIM, dim),
                        (1, 0, 2, 3)).reshape(NUM_HEADS, 3 * HEAD_DIM, dim)
    wq_pad = jnp.pad(wqr, ((0, 0), (0, HPAD - 3 * HEAD_DIM), (0, 0))
                     ).reshape(NUM_HEADS * HPAD, dim)
    gp_pad = jnp.pad(W_gp.reshape(NUM_HEADS, GP_NUM, HEAD_DIM),
                     ((0, 0), (0, 0), (0, HPAD - HEAD_DIM)))
    wp_h = jnp.transpose(W_proj.reshape(dim, NUM_HEADS, HEAD_DIM), (1, 2, 0))
    wp_pad = jnp.pad(wp_h, ((0, 0), (2 * HEAD_DIM, HPAD - 3 * HEAD_DIM),
                            (0, 0))).astype(jnp.bfloat16)

    qkv, oh, gm = pl.pallas_call(
        functools.partial(_route_body, b),
        grid=(b + 1,),
        in_specs=[
            pl.BlockSpec((1, n, dim), lambda i: (jnp.minimum(i, b - 1), 0, 0)),
            pl.BlockSpec((NUM_HEADS * HPAD, dim), lambda i: (0, 0)),
            pl.BlockSpec((NUM_HEADS, GP_NUM, HPAD), lambda i: (0, 0, 0)),
        ],
        out_specs=[
            pl.BlockSpec((1, n, NUM_HEADS * HPAD),
                         lambda i: (jnp.minimum(i, b - 1), 0, 0)),
            pl.BlockSpec((1, NUM_HEADS, GP_NUM, n),
                         lambda i: (jnp.minimum(i, b - 1), 0, 0, 0)),
            pl.BlockSpec((1, NUM_HEADS, GP_NUM, n),
                         lambda i: (jnp.maximum(i - 1, 0), 0, 0, 0)),
        ],
        out_shape=[
            jax.ShapeDtypeStruct((b, n, NUM_HEADS * HPAD), jnp.bfloat16),
            jax.ShapeDtypeStruct((b, NUM_HEADS, GP_NUM, n), jnp.bfloat16),
            jax.ShapeDtypeStruct((b, NUM_HEADS, GP_NUM, n), jnp.bfloat16),
        ],
        scratch_shapes=[
            pltpu.VMEM((2, NUM_HEADS * GP_NUM, n), jnp.int32),
        ],
        compiler_params=pltpu.CompilerParams(
            dimension_semantics=("arbitrary",),
        ),
    )(x3, wq_pad, gp_pad)

    out = pl.pallas_call(
        _attn_body,
        grid=(b, NUM_HEADS),
        in_specs=[
            pl.BlockSpec((1, n, HPAD), lambda i, j: (i, 0, j)),
            pl.BlockSpec((1, 1, GP_NUM, n), lambda i, j: (i, j, 0, 0)),
            pl.BlockSpec((1, 1, GP_NUM, n), lambda i, j: (i, j, 0, 0)),
            pl.BlockSpec((1, HPAD, dim), lambda i, j: (j, 0, 0)),
        ],
        out_specs=pl.BlockSpec((1, n, dim), lambda i, j: (i, 0, 0)),
        out_shape=jax.ShapeDtypeStruct((b, n, dim), jnp.float32),
        compiler_params=pltpu.CompilerParams(
            dimension_semantics=("arbitrary", "arbitrary"),
        ),
    )(qkv, oh, gm, wp_pad)
    return out.reshape(b, hh, ww, dim)
